# baseline (speedup 1.0000x reference)
import functools

import jax
import jax.numpy as jnp
from jax import lax
from jax.experimental import pallas as pl
from jax.experimental.pallas import tpu as pltpu
from jax.experimental.pallas import tpu_sc as plsc

_TC_TOK = 16384
_DC = 512


def _argmin_body(n_dblk, x_ref, dict_ref, out_ref, bestv, besti):
    d = pl.program_id(1)
    dict_blk = dict_ref[...]
    xaug = x_ref[...]
    ds = (dict_blk * -2.0).astype(jnp.bfloat16)
    norms = jnp.sum(dict_blk * dict_blk, axis=1, keepdims=True)
    hi = norms.astype(jnp.bfloat16)
    r1 = norms - hi.astype(jnp.float32)
    mid = r1.astype(jnp.bfloat16)
    lo = (r1 - mid.astype(jnp.float32)).astype(jnp.bfloat16)
    lhs = jnp.concatenate([ds, hi, mid, lo], axis=1)
    dist = lax.dot_general(
        lhs, xaug, (((1,), (0,)), ((), ())),
        preferred_element_type=jnp.float32,
    )
    rows = jnp.broadcast_to(
        lax.broadcasted_iota(jnp.int32, (_DC, 1), 0).astype(jnp.float32),
        dist.shape)
    lmin = jnp.min(dist, axis=0, keepdims=True)
    larg = jnp.min(jnp.where(dist == lmin, rows, jnp.float32(1e9)),
                   axis=0, keepdims=True) + jnp.float32(d * _DC)

    @pl.when(d == 0)
    def _():
        bestv[...] = lmin
        besti[...] = larg

    @pl.when(d != 0)
    def _():
        upd = lmin < bestv[...]
        besti[...] = jnp.where(upd, larg, besti[...])
        bestv[...] = jnp.where(upd, lmin, bestv[...])

    @pl.when(d == n_dblk - 1)
    def _():
        out_ref[...] = besti[...].astype(jnp.int32).reshape(1, 1, _TC_TOK)


def _argmin_codes(xaug, dictionary):
    caug, t = xaug.shape
    d_codes, c = dictionary.shape
    n_tblk = t // _TC_TOK
    n_dblk = d_codes // _DC
    out = pl.pallas_call(
        functools.partial(_argmin_body, n_dblk),
        grid=(n_tblk, n_dblk),
        in_specs=[
            pl.BlockSpec((caug, _TC_TOK), lambda tb, db: (0, tb)),
            pl.BlockSpec((_DC, c), lambda tb, db: (db, 0)),
        ],
        out_specs=pl.BlockSpec((1, 1, _TC_TOK), lambda tb, db: (tb, 0, 0)),
        out_shape=jax.ShapeDtypeStruct((n_tblk, 1, _TC_TOK), jnp.int32),
        scratch_shapes=[
            pltpu.VMEM((1, _TC_TOK), jnp.float32),
            pltpu.VMEM((1, _TC_TOK), jnp.float32),
        ],
    )(xaug, dictionary)
    return out.reshape(t)


def _gather_rows(dictionary, idx_flat):
    b = idx_flat.shape[0]
    d_model = dictionary.shape[1]
    info = plsc.get_sparse_core_info()
    nw = info.num_cores * info.num_subcores
    b_per_w = b // nw
    n_sub = b_per_w // 128
    mesh = plsc.VectorSubcoreMesh(core_axis_name="c", subcore_axis_name="s")

    @functools.partial(
        pl.kernel,
        mesh=mesh,
        out_type=jax.ShapeDtypeStruct((b, d_model), jnp.float32),
        scratch_types=[
            pltpu.VMEM((b_per_w,), jnp.int32),
            pltpu.VMEM((b_per_w, d_model), jnp.float32),
            pltpu.SemaphoreType.DMA,
        ],
        compiler_params=pltpu.CompilerParams(use_tc_tiling_on_sc=False),
    )
    def gk(table_hbm, idx_hbm, out_hbm, idx_v, rows_v, sem):
        wid = lax.axis_index("s") * info.num_cores + lax.axis_index("c")
        base = wid * b_per_w
        pltpu.sync_copy(idx_hbm.at[pl.ds(base, b_per_w)], idx_v)
        copies = []
        for j in range(n_sub):
            copies.append(pltpu.async_copy(
                table_hbm.at[idx_v.at[pl.ds(j * 128, 128)]],
                rows_v.at[pl.ds(j * 128, 128), :],
                sem,
            ))
        for cp in copies:
            cp.wait()
        pltpu.sync_copy(rows_v, out_hbm.at[pl.ds(base, b_per_w)])

    return gk(dictionary, idx_flat)


def _prep(x, n, c, hw):
    x_cols = x.reshape(n, c, hw).transpose(1, 0, 2).reshape(c, n * hw)
    return jnp.concatenate(
        [x_cols.astype(jnp.bfloat16), jnp.ones((3, n * hw), jnp.bfloat16)],
        axis=0)


def kernel(inputs, inputs_thermal, dictionary):
    n, c, h, w = inputs.shape
    hw = h * w

    idx1 = _argmin_codes(_prep(inputs, n, c, hw), dictionary)
    emb1 = _gather_rows(dictionary, idx1)
    idx2 = _argmin_codes(_prep(inputs_thermal, n, c, hw), dictionary)
    emb2 = _gather_rows(dictionary, idx2)

    def assemble(emb_flat):
        return (emb_flat.reshape(n, hw, c).transpose(0, 2, 1)
                .reshape(n, c, h, w))

    embedded = assemble(emb1)
    embedded_thermal = assemble(emb2)
    idxs = idx1.reshape(n, h, w)
    idxs_thermal = idx2.reshape(n, h, w)
    return (embedded, embedded, idxs, embedded_thermal, embedded_thermal,
            idxs_thermal)


# speedup vs baseline: 1.1421x; 1.1421x over previous
import functools

import jax
import jax.numpy as jnp
from jax import lax
from jax.experimental import pallas as pl
from jax.experimental.pallas import tpu as pltpu
from jax.experimental.pallas import tpu_sc as plsc

_TC_TOK = 16384
_DC = 256


def _argmin_body(n_dblk, x_ref, dict_ref, out_ref, bestv, besti):
    d = pl.program_id(1)
    dict_blk = dict_ref[...]
    xaug = x_ref[...]
    ds = (dict_blk * -2.0).astype(jnp.bfloat16)
    norms = jnp.sum(dict_blk * dict_blk, axis=1, keepdims=True)
    hi = norms.astype(jnp.bfloat16)
    r1 = norms - hi.astype(jnp.float32)
    mid = r1.astype(jnp.bfloat16)
    lo = (r1 - mid.astype(jnp.float32)).astype(jnp.bfloat16)
    lhs = jnp.concatenate([ds, hi, mid, lo], axis=1)
    dist = lax.dot_general(
        lhs, xaug, (((1,), (0,)), ((), ())),
        preferred_element_type=jnp.float32,
    )
    rows = jnp.broadcast_to(
        lax.broadcasted_iota(jnp.int32, (_DC, 1), 0).astype(jnp.float32),
        dist.shape)
    lmin = jnp.min(dist, axis=0, keepdims=True)
    larg = jnp.min(jnp.where(dist == lmin, rows, jnp.float32(1e9)),
                   axis=0, keepdims=True) + jnp.float32(d * _DC)

    @pl.when(d == 0)
    def _():
        bestv[...] = lmin
        besti[...] = larg

    @pl.when(d != 0)
    def _():
        upd = lmin < bestv[...]
        besti[...] = jnp.where(upd, larg, besti[...])
        bestv[...] = jnp.where(upd, lmin, bestv[...])

    @pl.when(d == n_dblk - 1)
    def _():
        out_ref[...] = besti[...].astype(jnp.int32).reshape(1, 1, _TC_TOK)


def _argmin_codes(xaug, dictionary):
    caug, t = xaug.shape
    d_codes, c = dictionary.shape
    n_tblk = t // _TC_TOK
    n_dblk = d_codes // _DC
    out = pl.pallas_call(
        functools.partial(_argmin_body, n_dblk),
        grid=(n_tblk, n_dblk),
        in_specs=[
            pl.BlockSpec((caug, _TC_TOK), lambda tb, db: (0, tb)),
            pl.BlockSpec((_DC, c), lambda tb, db: (db, 0)),
        ],
        out_specs=pl.BlockSpec((1, 1, _TC_TOK), lambda tb, db: (tb, 0, 0)),
        out_shape=jax.ShapeDtypeStruct((n_tblk, 1, _TC_TOK), jnp.int32),
        scratch_shapes=[
            pltpu.VMEM((1, _TC_TOK), jnp.float32),
            pltpu.VMEM((1, _TC_TOK), jnp.float32),
        ],
    )(xaug, dictionary)
    return out.reshape(t)


def _gather_rows(dictionary, idx_flat):
    b = idx_flat.shape[0]
    d_model = dictionary.shape[1]
    info = plsc.get_sparse_core_info()
    nw = info.num_cores * info.num_subcores
    b_per_w = b // nw
    n_sub = b_per_w // 128
    mesh = plsc.VectorSubcoreMesh(core_axis_name="c", subcore_axis_name="s")

    @functools.partial(
        pl.kernel,
        mesh=mesh,
        out_type=jax.ShapeDtypeStruct((b, d_model), jnp.float32),
        scratch_types=[
            pltpu.VMEM((b_per_w,), jnp.int32),
            pltpu.VMEM((b_per_w, d_model), jnp.float32),
            pltpu.SemaphoreType.DMA,
        ],
        compiler_params=pltpu.CompilerParams(use_tc_tiling_on_sc=False),
    )
    def gk(table_hbm, idx_hbm, out_hbm, idx_v, rows_v, sem):
        wid = lax.axis_index("s") * info.num_cores + lax.axis_index("c")
        base = wid * b_per_w
        pltpu.sync_copy(idx_hbm.at[pl.ds(base, b_per_w)], idx_v)
        copies = []
        for j in range(n_sub):
            copies.append(pltpu.async_copy(
                table_hbm.at[idx_v.at[pl.ds(j * 128, 128)]],
                rows_v.at[pl.ds(j * 128, 128), :],
                sem,
            ))
        for cp in copies:
            cp.wait()
        pltpu.sync_copy(rows_v, out_hbm.at[pl.ds(base, b_per_w)])

    return gk(dictionary, idx_flat)


def _prep(x, n, c, hw):
    x_cols = x.reshape(n, c, hw).transpose(1, 0, 2).reshape(c, n * hw)
    return jnp.concatenate(
        [x_cols.astype(jnp.bfloat16), jnp.ones((3, n * hw), jnp.bfloat16)],
        axis=0)


def kernel(inputs, inputs_thermal, dictionary):
    n, c, h, w = inputs.shape
    hw = h * w

    idx1 = _argmin_codes(_prep(inputs, n, c, hw), dictionary)
    emb1 = _gather_rows(dictionary, idx1)
    idx2 = _argmin_codes(_prep(inputs_thermal, n, c, hw), dictionary)
    emb2 = _gather_rows(dictionary, idx2)

    def assemble(emb_flat):
        return (emb_flat.reshape(n, hw, c).transpose(0, 2, 1)
                .reshape(n, c, h, w))

    embedded = assemble(emb1)
    embedded_thermal = assemble(emb2)
    idxs = idx1.reshape(n, h, w)
    idxs_thermal = idx2.reshape(n, h, w)
    return (embedded, embedded, idxs, embedded_thermal, embedded_thermal,
            idxs_thermal)
